# baseline (speedup 1.0000x reference)
import functools

import jax
import jax.numpy as jnp
from jax import lax
from jax.experimental import pallas as pl
from jax.experimental.pallas import tpu as pltpu
from jax.experimental.pallas import tpu_sc as plsc

LANES = 128
SUB = 8
GRP = 8


def _build(B, H, V, D, num_cores, num_subcores):
    nw = num_cores * num_subcores
    n_units_total = (B // LANES) * H
    units_per_w = n_units_total // nw
    n_per_w = units_per_w * LANES
    d_tiles = D // SUB
    n_groups = units_per_w // GRP
    tc_blocks = B // LANES
    mesh = plsc.VectorSubcoreMesh(core_axis_name="c", subcore_axis_name="s")

    @functools.partial(
        pl.kernel,
        mesh=mesh,
        out_type=jax.ShapeDtypeStruct((H, d_tiles, tc_blocks, SUB, LANES),
                                      jnp.float32),
        scratch_types=[
            pltpu.VMEM((n_per_w,), jnp.int32),
            pltpu.VMEM((2, GRP, LANES, D), jnp.float32),
            pltpu.SemaphoreType.DMA,
            pltpu.SemaphoreType.DMA,
        ],
        compiler_params=pltpu.CompilerParams(use_tc_tiling_on_sc=False,
                                             needs_layout_passes=False),
    )
    def k(xf_hbm, table_hbm, po_hbm, idx_v, rows_v, gsem, osem):
        wid = lax.axis_index("s") * num_cores + lax.axis_index("c")
        u_base = wid * units_per_w

        pltpu.sync_copy(xf_hbm.at[pl.ds(wid * n_per_w, n_per_w)], idx_v)

        def gathers(g, r):
            off = lax.rem(g, n_groups) * (GRP * LANES)

            def fire(kk, c):
                pltpu.async_copy(
                    table_hbm.at[idx_v.at[pl.ds(off + kk * LANES, LANES)]],
                    rows_v.at[r, kk],
                    gsem,
                )
                return c

            lax.fori_loop(0, GRP, fire, 0)

        def drain_gathers(r):
            def drain(kk, c):
                pltpu.make_async_copy(
                    table_hbm.at[idx_v.at[pl.ds(0, LANES)]],
                    rows_v.at[r, kk],
                    gsem,
                ).wait()
                return c

            lax.fori_loop(0, GRP, drain, 0)

        def writes(g, r):
            u0 = u_base + g * GRP
            h = lax.div(u0, tc_blocks)
            tc0 = lax.rem(u0, tc_blocks)

            def fire(cc, c):
                tr = lax.div(cc, SUB)
                s = lax.rem(cc, SUB)
                pltpu.async_copy(
                    rows_v.at[r, pl.ds(0, GRP), pl.ds(0, LANES), cc],
                    po_hbm.at[h, tr, pl.ds(tc0, GRP), s],
                    osem,
                )
                return c

            lax.fori_loop(0, D, fire, 0)

        def drain_writes(r):
            def drain(cc, c):
                pltpu.make_async_copy(
                    rows_v.at[r, pl.ds(0, GRP), pl.ds(0, LANES), cc],
                    po_hbm.at[0, 0, pl.ds(0, GRP), 0],
                    osem,
                ).wait()
                return c

            lax.fori_loop(0, D, drain, 0)

        gathers(0, 0)
        drain_gathers(0)
        gathers(1, 1)
        writes(0, 0)

        def body(t, carry):
            for b2 in range(2):
                g = 1 + 2 * t + b2
                r = (1 + b2) % 2
                drain_writes(1 - r)
                drain_gathers(r)
                gathers(g + 1, 1 - r)
                writes(g, r)
            return carry

        lax.fori_loop(0, (n_groups - 1) // 2, body, 0)

        drain_gathers(1)
        drain_writes(0)

    return k


def kernel(x, table):
    B, H = x.shape
    V, D = table.shape
    info = plsc.get_sparse_core_info()
    k = _build(B, H, V, D, info.num_cores, info.num_subcores)
    xf = jnp.transpose(x).reshape(B * H)
    po = k(xf, table)
    out = jnp.transpose(po, (2, 4, 0, 1, 3)).reshape(B, H, D)
    return out


# speedup vs baseline: 70.3376x; 70.3376x over previous
import functools

import jax
import jax.numpy as jnp
from jax import lax
from jax.experimental import pallas as pl
from jax.experimental.pallas import tpu as pltpu
from jax.experimental.pallas import tpu_sc as plsc

LANES = 128
SUB = 8
NB = 2


def _build(B, H, V, D, num_cores, num_subcores):
    nw = num_cores * num_subcores
    n_units_total = (B // LANES) * H
    units_per_w = n_units_total // nw
    n_per_w = units_per_w * LANES
    d_tiles = D // SUB
    mesh = plsc.VectorSubcoreMesh(core_axis_name="c", subcore_axis_name="s")

    @functools.partial(
        pl.kernel,
        mesh=mesh,
        out_type=jax.ShapeDtypeStruct((H, d_tiles, B // LANES, SUB, LANES),
                                      jnp.float32),
        scratch_types=[
            pltpu.VMEM((n_per_w,), jnp.int32),
            pltpu.VMEM((NB, LANES, D), jnp.float32),
            pltpu.VMEM((NB, d_tiles, SUB, LANES), jnp.float32),
            pltpu.SemaphoreType.DMA,
            pltpu.SemaphoreType.DMA,
        ],
        compiler_params=pltpu.CompilerParams(use_tc_tiling_on_sc=False,
                                             needs_layout_passes=False),
    )
    def k(xf_hbm, table_hbm, po_hbm, idx_v, rows_v, tpo_v, gsem, osem):
        wid = lax.axis_index("s") * num_cores + lax.axis_index("c")
        u_base = wid * units_per_w

        pltpu.sync_copy(xf_hbm.at[pl.ds(wid * n_per_w, n_per_w)], idx_v)

        iota16 = lax.iota(jnp.int32, 16)
        riota = [iota16 + l0 for l0 in range(0, LANES, 16)]

        def gather(j, b):
            return pltpu.async_copy(
                table_hbm.at[idx_v.at[pl.ds(j * LANES, LANES)]],
                rows_v.at[b],
                gsem,
            )

        def transpose(b):
            @plsc.parallel_loop(0, D, unroll=4)
            def _(cc):
                tr = lax.div(cc, SUB)
                s = lax.rem(cc, SUB)
                col = jnp.full((16,), cc, jnp.int32)
                for li in range(LANES // 16):
                    v = plsc.load_gather(rows_v.at[b], [riota[li], col])
                    tpo_v[b, tr, s, pl.ds(li * 16, 16)] = v

        def writeback(j, b):
            u = u_base + j
            h = lax.div(u, B // LANES)
            tc = lax.rem(u, B // LANES)
            for tr in range(d_tiles):
                pltpu.async_copy(
                    tpo_v.at[b, tr], po_hbm.at[h, tr, tc], osem)

        def drain_gather(b):
            pltpu.make_async_copy(
                table_hbm.at[idx_v.at[pl.ds(0, LANES)]], rows_v.at[b], gsem
            ).wait()

        def drain_write(b):
            pltpu.make_async_copy(
                po_hbm.at[0, pl.ds(0, d_tiles), 0], tpo_v.at[b], osem
            ).wait()

        def step(j, b, fire_ahead, drain_prev):
            drain_gather(b)
            if drain_prev:
                drain_write(b)
            transpose(b)
            if fire_ahead:
                gather(j + NB, b)
            writeback(j, b)

        for b in range(NB):
            gather(b, b)
        for j in range(NB):
            step(j, j % NB, True, False)

        def body(t, carry):
            for bu in range(NB):
                j = NB + t * NB + bu
                step(j, bu, True, True)
            return carry

        n_mid = (units_per_w - 3 * NB) // NB
        lax.fori_loop(0, n_mid, body, 0)

        for j in range(units_per_w - 2 * NB, units_per_w):
            step(j, j % NB, j + NB < units_per_w, True)
        for b in range(NB):
            drain_write(b)

    return k


def kernel(x, table):
    B, H = x.shape
    V, D = table.shape
    info = plsc.get_sparse_core_info()
    k = _build(B, H, V, D, info.num_cores, info.num_subcores)
    xf = jnp.transpose(x).reshape(B * H)
    po = k(xf, table)
    out = jnp.transpose(po, (2, 4, 0, 1, 3)).reshape(B, H, D)
    return out
